# baseline (speedup 1.0000x reference)
import functools

import jax
import jax.numpy as jnp
from jax import lax
from jax.experimental import pallas as pl
from jax.experimental.pallas import tpu as pltpu
from jax.experimental.pallas import tpu_sc as plsc

N = 10000
E = 320000
IN_DIM = 128
HID_DIM = 256
OUT_DIM = 2

XA_D = 144
PQ_D = 16
NC = 2
NS = 16
EPT = E // (NC * NS)
CH = 128
NCH = 80
EPT_P = NCH * CH
NP = N + 8
NSEG = 10
NCH_SEG = NCH // NSEG
NB2 = 8
NR2 = NCH // NB2


def _sc_aggregate(table, src, dst, zeros, d):
    mesh = plsc.VectorSubcoreMesh(core_axis_name="core", subcore_axis_name="subcore")

    @functools.partial(
        pl.kernel,
        out_type=jax.ShapeDtypeStruct((NC, NP, d), jnp.float32),
        mesh=mesh,
        compiler_params=pltpu.CompilerParams(use_tc_tiling_on_sc=False),
        scratch_types=[
            pltpu.VMEM((NCH_SEG, CH), jnp.int32),
            pltpu.VMEM((NCH_SEG, CH), jnp.int32),
            pltpu.VMEM((CH, d), jnp.float32),
            pltpu.VMEM((CH, d), jnp.float32),
            pltpu.SemaphoreType.DMA,
            pltpu.SemaphoreType.DMA,
            pltpu.VMEM_SHARED((NP, d), jnp.float32),
        ],
    )
    def agg_kernel(table_hbm, src_hbm, dst_hbm, zeros_hbm, out_hbm,
                   srcs, dsts, rows0, rows1, sem0, sem1, acc):
        c = lax.axis_index("core")
        s = lax.axis_index("subcore")

        @pl.when(s == 0)
        def _():
            pltpu.sync_copy(zeros_hbm, acc)

        plsc.subcore_barrier()

        @pl.loop(0, NSEG)
        def _(g):
            pltpu.sync_copy(src_hbm.at[c, s, pl.ds(g * NCH_SEG, NCH_SEG)], srcs)
            pltpu.sync_copy(dst_hbm.at[c, s, pl.ds(g * NCH_SEG, NCH_SEG)], dsts)

            pltpu.async_copy(table_hbm.at[srcs.at[0]], rows0, sem0)

            @pl.loop(0, NCH_SEG, step=2)
            def _(i):
                pltpu.make_async_copy(table_hbm.at[srcs.at[i]], rows0, sem0).wait()
                pltpu.async_copy(table_hbm.at[srcs.at[i + 1]], rows1, sem1)
                pltpu.sync_copy(rows0, acc.at[dsts.at[i]], add=True)
                pltpu.make_async_copy(table_hbm.at[srcs.at[i + 1]], rows1,
                                      sem1).wait()

                @pl.when(i + 2 < NCH_SEG)
                def _():
                    pltpu.async_copy(table_hbm.at[srcs.at[i + 2]], rows0, sem0)

                pltpu.sync_copy(rows1, acc.at[dsts.at[i + 1]], add=True)

        plsc.subcore_barrier()

        rpt = 624
        r0 = s * rpt
        pltpu.sync_copy(acc.at[pl.ds(r0, rpt)], out_hbm.at[c, pl.ds(r0, rpt)])

        @pl.when(s == 0)
        def _():
            tail = NS * rpt
            pltpu.sync_copy(acc.at[pl.ds(tail, NP - tail)],
                            out_hbm.at[c, pl.ds(tail, NP - tail)])

    return agg_kernel(table, src, dst, zeros)


def _sc_aggregate_small(table, src, dst, zeros, d):
    mesh = plsc.VectorSubcoreMesh(core_axis_name="core", subcore_axis_name="subcore")

    @functools.partial(
        pl.kernel,
        out_type=jax.ShapeDtypeStruct((NC, NP, d), jnp.float32),
        mesh=mesh,
        compiler_params=pltpu.CompilerParams(use_tc_tiling_on_sc=False),
        scratch_types=(
            [pltpu.VMEM((NCH, CH), jnp.int32)] * 2
            + [pltpu.VMEM((CH, d), jnp.float32)] * NB2
            + [pltpu.SemaphoreType.DMA] * (2 * NB2)
            + [pltpu.VMEM_SHARED((NP, d), jnp.float32)]
        ),
    )
    def agg_kernel(table_hbm, src_hbm, dst_hbm, zeros_hbm, out_hbm, *scr):
        srcs, dsts = scr[0], scr[1]
        rows = scr[2:2 + NB2]
        gsem = scr[2 + NB2:2 + 2 * NB2]
        ssem = scr[2 + 2 * NB2:2 + 3 * NB2]
        acc = scr[2 + 3 * NB2]
        c = lax.axis_index("core")
        s = lax.axis_index("subcore")

        @pl.when(s == 0)
        def _():
            pltpu.sync_copy(zeros_hbm, acc)

        pltpu.sync_copy(src_hbm.at[c, s], srcs)
        pltpu.sync_copy(dst_hbm.at[c, s], dsts)

        plsc.subcore_barrier()

        for j in range(NB2):
            pltpu.async_copy(table_hbm.at[srcs.at[j]], rows[j], gsem[j])

        @pl.loop(0, NR2)
        def _(r):
            base = r * NB2
            for j in range(NB2):
                pltpu.make_async_copy(table_hbm.at[srcs.at[base + j]],
                                      rows[j], gsem[j]).wait()
                pltpu.async_copy(rows[j], acc.at[dsts.at[base + j]], ssem[j],
                                 add=True)

            @pl.when(r < NR2 - 1)
            def _():
                for j in range(NB2):
                    pltpu.make_async_copy(rows[j], acc.at[dsts.at[base + j]],
                                          ssem[j]).wait()
                    pltpu.async_copy(table_hbm.at[srcs.at[base + NB2 + j]],
                                     rows[j], gsem[j])

        for j in range(NB2):
            pltpu.make_async_copy(rows[j], acc.at[dsts.at[NCH - NB2 + j]],
                                  ssem[j]).wait()

        plsc.subcore_barrier()

        rpt = 624
        r0 = s * rpt
        pltpu.sync_copy(acc.at[pl.ds(r0, rpt)], out_hbm.at[c, pl.ds(r0, rpt)])

        @pl.when(s == 0)
        def _():
            tail = NS * rpt
            pltpu.sync_copy(acc.at[pl.ds(tail, NP - tail)],
                            out_hbm.at[c, pl.ds(tail, NP - tail)])

    return agg_kernel(table, src, dst, zeros)


BR = 2000
NBR = N // BR


def _dot(a, b):
    return lax.dot_general(a, b, (((1,), (0,)), ((), ())),
                           preferred_element_type=jnp.float32,
                           precision=lax.Precision.DEFAULT)


def _tc_sage1(aggp, x, w1lt, w1rt, b1l2):
    def body(ap_ref, x_ref, w1l_ref, w1r_ref, b1l_ref,
             hpre_ref, civ_ref, stats_ref):
        i = pl.program_id(0)
        agg = ap_ref[0] + ap_ref[1]
        civ = 1.0 / jnp.maximum(agg[:, IN_DIM:IN_DIM + 1], 1.0)
        mean = agg[:, :IN_DIM] * civ
        h = _dot(mean, w1l_ref[...]) + _dot(x_ref[...], w1r_ref[...]) + b1l_ref[...]
        hpre_ref[...] = h
        civ_ref[...] = civ

        @pl.when(i == 0)
        def _():
            stats_ref[...] = jnp.zeros_like(stats_ref)

        stats_ref[0:1, :] += jnp.sum(h, axis=0, keepdims=True)
        stats_ref[1:2, :] += jnp.sum(h * h, axis=0, keepdims=True)

    return pl.pallas_call(
        body,
        grid=(NBR,),
        in_specs=[
            pl.BlockSpec((NC, BR, XA_D), lambda i: (0, i, 0)),
            pl.BlockSpec((BR, IN_DIM), lambda i: (i, 0)),
            pl.BlockSpec((IN_DIM, HID_DIM), lambda i: (0, 0)),
            pl.BlockSpec((IN_DIM, HID_DIM), lambda i: (0, 0)),
            pl.BlockSpec((1, HID_DIM), lambda i: (0, 0)),
        ],
        out_specs=[
            pl.BlockSpec((BR, HID_DIM), lambda i: (i, 0)),
            pl.BlockSpec((BR, 1), lambda i: (i, 0)),
            pl.BlockSpec((2, HID_DIM), lambda i: (0, 0)),
        ],
        out_shape=[
            jax.ShapeDtypeStruct((N, HID_DIM), jnp.float32),
            jax.ShapeDtypeStruct((N, 1), jnp.float32),
            jax.ShapeDtypeStruct((2, HID_DIM), jnp.float32),
        ],
    )(aggp, x, w1lt, w1rt, b1l2)


def _tc_bn_proj(hpre, civ, stats, gamma2, beta2, w2t):
    def body(h_ref, civ_ref, stats_ref, g_ref, bta_ref, w2t_ref, pq_ref):
        mu = stats_ref[0:1, :] * (1.0 / N)
        var = stats_ref[1:2, :] * (1.0 / N) - mu * mu
        h = (h_ref[...] - mu) * lax.rsqrt(var + 1e-5) * g_ref[...] + bta_ref[...]
        h = jnp.maximum(h, 0.0)
        pq = _dot(h, w2t_ref[...])
        pq_ref[...] = jnp.concatenate(
            [pq[:, :2 * OUT_DIM], civ_ref[...], pq[:, 2 * OUT_DIM + 1:]], axis=1)

    return pl.pallas_call(
        body,
        grid=(NBR,),
        in_specs=[
            pl.BlockSpec((BR, HID_DIM), lambda i: (i, 0)),
            pl.BlockSpec((BR, 1), lambda i: (i, 0)),
            pl.BlockSpec((2, HID_DIM), lambda i: (0, 0)),
            pl.BlockSpec((1, HID_DIM), lambda i: (0, 0)),
            pl.BlockSpec((1, HID_DIM), lambda i: (0, 0)),
            pl.BlockSpec((HID_DIM, PQ_D), lambda i: (0, 0)),
        ],
        out_specs=pl.BlockSpec((BR, PQ_D), lambda i: (i, 0)),
        out_shape=jax.ShapeDtypeStruct((N, PQ_D), jnp.float32),
    )(hpre, civ, stats, gamma2, beta2, w2t)


def _tc_final(agg2p, pq, b2l2):
    def body(a_ref, pq_ref, b_ref, out_ref):
        a = a_ref[0] + a_ref[1]
        meanp = a[:, :OUT_DIM] * pq_ref[:, 2 * OUT_DIM:2 * OUT_DIM + 1]
        out_ref[...] = meanp + pq_ref[:, OUT_DIM:2 * OUT_DIM] + b_ref[...]

    return pl.pallas_call(
        body,
        grid=(NBR,),
        in_specs=[
            pl.BlockSpec((NC, BR, PQ_D), lambda i: (0, i, 0)),
            pl.BlockSpec((BR, PQ_D), lambda i: (i, 0)),
            pl.BlockSpec((1, OUT_DIM), lambda i: (0, 0)),
        ],
        out_specs=pl.BlockSpec((BR, OUT_DIM), lambda i: (i, 0)),
        out_shape=jax.ShapeDtypeStruct((N, OUT_DIM), jnp.float32),
    )(agg2p, pq, b2l2)


def kernel(x, edge_index, W1l, b1l, W1r, gamma1, beta1, W2l, b2l, W2r):
    pad_s = jnp.zeros((NC * NS, EPT_P - EPT), jnp.int32)
    pad_d = jnp.full((NC * NS, EPT_P - EPT), N, jnp.int32)
    src_p = jnp.concatenate([edge_index[0].reshape(NC * NS, EPT), pad_s],
                            axis=1).reshape(NC, NS, NCH, CH)
    dst_p = jnp.concatenate([edge_index[1].reshape(NC * NS, EPT), pad_d],
                            axis=1).reshape(NC, NS, NCH, CH)

    xa = jnp.concatenate(
        [x, jnp.ones((N, 1), jnp.float32),
         jnp.zeros((N, XA_D - IN_DIM - 1), jnp.float32)], axis=1)
    zeros1 = jnp.zeros((NP, XA_D), jnp.float32)
    aggp = _sc_aggregate(xa, src_p, dst_p, zeros1, XA_D)

    w2t = jnp.concatenate(
        [W2l.T, W2r.T, jnp.zeros((HID_DIM, PQ_D - 2 * OUT_DIM), jnp.float32)],
        axis=1)
    hpre, civ, stats = _tc_sage1(aggp, x, W1l.T, W1r.T, b1l.reshape(1, -1))
    pq = _tc_bn_proj(hpre, civ, stats, gamma1.reshape(1, -1),
                     beta1.reshape(1, -1), w2t)

    zeros2 = jnp.zeros((NP, PQ_D), jnp.float32)
    agg2p = _sc_aggregate_small(pq, src_p, dst_p, zeros2, PQ_D)

    return _tc_final(agg2p, pq, b2l.reshape(1, -1))


# speedup vs baseline: 1.7948x; 1.7948x over previous
import functools

import jax
import jax.numpy as jnp
from jax import lax
from jax.experimental import pallas as pl
from jax.experimental.pallas import tpu as pltpu
from jax.experimental.pallas import tpu_sc as plsc

N = 10000
E = 320000
IN_DIM = 128
HID_DIM = 256
OUT_DIM = 2

XA_D = 144
PQ_D = 16
NC = 2
NS = 16
EPT = E // (NC * NS)
CH = 128
NCH = 80
EPT_P = NCH * CH
NP = N + 8
NB2 = 8
NR2 = NCH // NB2
CH1 = 80
NCHUNK1 = EPT // CH1
NSEG1 = 5
NCH_SEG1 = NCHUNK1 // NSEG1


def _sc_aggregate(table, src, dst, zeros, d):
    mesh = plsc.VectorSubcoreMesh(core_axis_name="core", subcore_axis_name="subcore")

    @functools.partial(
        pl.kernel,
        out_type=jax.ShapeDtypeStruct((NC, N, d), jnp.float32),
        mesh=mesh,
        compiler_params=pltpu.CompilerParams(use_tc_tiling_on_sc=False),
        scratch_types=[
            pltpu.VMEM((NCH_SEG1, CH1), jnp.int32),
            pltpu.VMEM((NCH_SEG1, CH1), jnp.int32),
            pltpu.VMEM((CH1, d), jnp.float32),
            pltpu.VMEM((CH1, d), jnp.float32),
            pltpu.SemaphoreType.DMA,
            pltpu.SemaphoreType.DMA,
            pltpu.VMEM_SHARED((N, d), jnp.float32),
        ],
    )
    def agg_kernel(table_hbm, src_hbm, dst_hbm, zeros_hbm, out_hbm,
                   srcs, dsts, rows0, rows1, sem0, sem1, acc):
        c = lax.axis_index("core")
        s = lax.axis_index("subcore")

        @pl.when(s == 0)
        def _():
            pltpu.sync_copy(zeros_hbm, acc)

        plsc.subcore_barrier()

        @pl.loop(0, NSEG1)
        def _(g):
            pltpu.sync_copy(src_hbm.at[c, s, g], srcs)
            pltpu.sync_copy(dst_hbm.at[c, s, g], dsts)

            pltpu.async_copy(table_hbm.at[srcs.at[0]], rows0, sem0)

            @pl.loop(0, NCH_SEG1 - 1, step=2)
            def _(i):
                pltpu.make_async_copy(table_hbm.at[srcs.at[i]], rows0, sem0).wait()
                pltpu.async_copy(table_hbm.at[srcs.at[i + 1]], rows1, sem1)
                pltpu.sync_copy(rows0, acc.at[dsts.at[i]], add=True)
                pltpu.make_async_copy(table_hbm.at[srcs.at[i + 1]], rows1,
                                      sem1).wait()

                @pl.when(i + 2 < NCH_SEG1)
                def _():
                    pltpu.async_copy(table_hbm.at[srcs.at[i + 2]], rows0, sem0)

                pltpu.sync_copy(rows1, acc.at[dsts.at[i + 1]], add=True)

            pltpu.make_async_copy(table_hbm.at[srcs.at[NCH_SEG1 - 1]], rows0,
                                  sem0).wait()
            pltpu.sync_copy(rows0, acc.at[dsts.at[NCH_SEG1 - 1]], add=True)

        plsc.subcore_barrier()

        rpt = 624
        r0 = s * rpt
        pltpu.sync_copy(acc.at[pl.ds(r0, rpt)], out_hbm.at[c, pl.ds(r0, rpt)])

        @pl.when(s == 0)
        def _():
            tail = NS * rpt
            pltpu.sync_copy(acc.at[pl.ds(tail, N - tail)],
                            out_hbm.at[c, pl.ds(tail, N - tail)])

    return agg_kernel(table, src, dst, zeros)


def _sc_aggregate_small(table, src, dst, zeros, d):
    mesh = plsc.VectorSubcoreMesh(core_axis_name="core", subcore_axis_name="subcore")

    @functools.partial(
        pl.kernel,
        out_type=jax.ShapeDtypeStruct((NC, NP, d), jnp.float32),
        mesh=mesh,
        compiler_params=pltpu.CompilerParams(use_tc_tiling_on_sc=False),
        scratch_types=(
            [pltpu.VMEM((NCH, CH), jnp.int32)] * 2
            + [pltpu.VMEM((CH, d), jnp.float32)] * NB2
            + [pltpu.SemaphoreType.DMA] * (2 * NB2)
            + [pltpu.VMEM_SHARED((NP, d), jnp.float32)]
        ),
    )
    def agg_kernel(table_hbm, src_hbm, dst_hbm, zeros_hbm, out_hbm, *scr):
        srcs, dsts = scr[0], scr[1]
        rows = scr[2:2 + NB2]
        gsem = scr[2 + NB2:2 + 2 * NB2]
        ssem = scr[2 + 2 * NB2:2 + 3 * NB2]
        acc = scr[2 + 3 * NB2]
        c = lax.axis_index("core")
        s = lax.axis_index("subcore")

        @pl.when(s == 0)
        def _():
            pltpu.sync_copy(zeros_hbm, acc)

        pltpu.sync_copy(src_hbm.at[c, s], srcs)
        pltpu.sync_copy(dst_hbm.at[c, s], dsts)

        plsc.subcore_barrier()

        for j in range(NB2):
            pltpu.async_copy(table_hbm.at[srcs.at[j]], rows[j], gsem[j])

        @pl.loop(0, NR2)
        def _(r):
            base = r * NB2
            for j in range(NB2):
                pltpu.make_async_copy(table_hbm.at[srcs.at[base + j]],
                                      rows[j], gsem[j]).wait()
                pltpu.async_copy(rows[j], acc.at[dsts.at[base + j]], ssem[j],
                                 add=True)

            @pl.when(r < NR2 - 1)
            def _():
                for j in range(NB2):
                    pltpu.make_async_copy(rows[j], acc.at[dsts.at[base + j]],
                                          ssem[j]).wait()
                    pltpu.async_copy(table_hbm.at[srcs.at[base + NB2 + j]],
                                     rows[j], gsem[j])

        for j in range(NB2):
            pltpu.make_async_copy(rows[j], acc.at[dsts.at[NCH - NB2 + j]],
                                  ssem[j]).wait()

        plsc.subcore_barrier()

        rpt = 624
        r0 = s * rpt
        pltpu.sync_copy(acc.at[pl.ds(r0, rpt)], out_hbm.at[c, pl.ds(r0, rpt)])

        @pl.when(s == 0)
        def _():
            tail = NS * rpt
            pltpu.sync_copy(acc.at[pl.ds(tail, NP - tail)],
                            out_hbm.at[c, pl.ds(tail, NP - tail)])

    return agg_kernel(table, src, dst, zeros)


BR = 2000
NBR = N // BR


def _dot(a, b):
    return lax.dot_general(a, b, (((1,), (0,)), ((), ())),
                           preferred_element_type=jnp.float32,
                           precision=lax.Precision.DEFAULT)


def _tc_sage1(aggp, x, w1lt, w1rt, b1l2):
    def body(ap_ref, x_ref, w1l_ref, w1r_ref, b1l_ref,
             hpre_ref, civ_ref, stats_ref):
        i = pl.program_id(0)
        agg = ap_ref[0] + ap_ref[1]
        civ = 1.0 / jnp.maximum(agg[:, IN_DIM:IN_DIM + 1], 1.0)
        mean = agg[:, :IN_DIM] * civ
        h = _dot(mean, w1l_ref[...]) + _dot(x_ref[...], w1r_ref[...]) + b1l_ref[...]
        hpre_ref[...] = h
        civ_ref[...] = civ

        @pl.when(i == 0)
        def _():
            stats_ref[...] = jnp.zeros_like(stats_ref)

        stats_ref[0:1, :] += jnp.sum(h, axis=0, keepdims=True)
        stats_ref[1:2, :] += jnp.sum(h * h, axis=0, keepdims=True)

    return pl.pallas_call(
        body,
        grid=(NBR,),
        in_specs=[
            pl.BlockSpec((NC, BR, XA_D), lambda i: (0, i, 0)),
            pl.BlockSpec((BR, IN_DIM), lambda i: (i, 0)),
            pl.BlockSpec((IN_DIM, HID_DIM), lambda i: (0, 0)),
            pl.BlockSpec((IN_DIM, HID_DIM), lambda i: (0, 0)),
            pl.BlockSpec((1, HID_DIM), lambda i: (0, 0)),
        ],
        out_specs=[
            pl.BlockSpec((BR, HID_DIM), lambda i: (i, 0)),
            pl.BlockSpec((BR, 1), lambda i: (i, 0)),
            pl.BlockSpec((2, HID_DIM), lambda i: (0, 0)),
        ],
        out_shape=[
            jax.ShapeDtypeStruct((N, HID_DIM), jnp.float32),
            jax.ShapeDtypeStruct((N, 1), jnp.float32),
            jax.ShapeDtypeStruct((2, HID_DIM), jnp.float32),
        ],
    )(aggp, x, w1lt, w1rt, b1l2)


def _tc_bn_proj(hpre, civ, stats, gamma2, beta2, w2t):
    def body(h_ref, civ_ref, stats_ref, g_ref, bta_ref, w2t_ref, pq_ref):
        mu = stats_ref[0:1, :] * (1.0 / N)
        var = stats_ref[1:2, :] * (1.0 / N) - mu * mu
        h = (h_ref[...] - mu) * lax.rsqrt(var + 1e-5) * g_ref[...] + bta_ref[...]
        h = jnp.maximum(h, 0.0)
        pq = _dot(h, w2t_ref[...])
        pq_ref[...] = jnp.concatenate(
            [pq[:, :2 * OUT_DIM], civ_ref[...], pq[:, 2 * OUT_DIM + 1:]], axis=1)

    return pl.pallas_call(
        body,
        grid=(NBR,),
        in_specs=[
            pl.BlockSpec((BR, HID_DIM), lambda i: (i, 0)),
            pl.BlockSpec((BR, 1), lambda i: (i, 0)),
            pl.BlockSpec((2, HID_DIM), lambda i: (0, 0)),
            pl.BlockSpec((1, HID_DIM), lambda i: (0, 0)),
            pl.BlockSpec((1, HID_DIM), lambda i: (0, 0)),
            pl.BlockSpec((HID_DIM, PQ_D), lambda i: (0, 0)),
        ],
        out_specs=pl.BlockSpec((BR, PQ_D), lambda i: (i, 0)),
        out_shape=jax.ShapeDtypeStruct((N, PQ_D), jnp.float32),
    )(hpre, civ, stats, gamma2, beta2, w2t)


def _tc_final(agg2p, pq, b2l2):
    def body(a_ref, pq_ref, b_ref, out_ref):
        a = a_ref[0] + a_ref[1]
        meanp = a[:, :OUT_DIM] * pq_ref[:, 2 * OUT_DIM:2 * OUT_DIM + 1]
        out_ref[...] = meanp + pq_ref[:, OUT_DIM:2 * OUT_DIM] + b_ref[...]

    return pl.pallas_call(
        body,
        grid=(NBR,),
        in_specs=[
            pl.BlockSpec((NC, BR, PQ_D), lambda i: (0, i, 0)),
            pl.BlockSpec((BR, PQ_D), lambda i: (i, 0)),
            pl.BlockSpec((1, OUT_DIM), lambda i: (0, 0)),
        ],
        out_specs=pl.BlockSpec((BR, OUT_DIM), lambda i: (i, 0)),
        out_shape=jax.ShapeDtypeStruct((N, OUT_DIM), jnp.float32),
    )(agg2p, pq, b2l2)


def kernel(x, edge_index, W1l, b1l, W1r, gamma1, beta1, W2l, b2l, W2r):
    pad_s = jnp.zeros((NC * NS, EPT_P - EPT), jnp.int32)
    pad_d = jnp.full((NC * NS, EPT_P - EPT), N, jnp.int32)
    src_p = jnp.concatenate([edge_index[0].reshape(NC * NS, EPT), pad_s],
                            axis=1).reshape(NC, NS, NCH, CH)
    dst_p = jnp.concatenate([edge_index[1].reshape(NC * NS, EPT), pad_d],
                            axis=1).reshape(NC, NS, NCH, CH)

    src_5 = edge_index[0].reshape(NC, NS, NSEG1, NCH_SEG1, CH1)
    dst_5 = edge_index[1].reshape(NC, NS, NSEG1, NCH_SEG1, CH1)
    xa = jnp.concatenate(
        [x, jnp.ones((N, 1), jnp.float32),
         jnp.zeros((N, XA_D - IN_DIM - 1), jnp.float32)], axis=1)
    zeros1 = jnp.zeros((N, XA_D), jnp.float32)
    aggp = _sc_aggregate(xa, src_5, dst_5, zeros1, XA_D)

    w2t = jnp.concatenate(
        [W2l.T, W2r.T, jnp.zeros((HID_DIM, PQ_D - 2 * OUT_DIM), jnp.float32)],
        axis=1)
    hpre, civ, stats = _tc_sage1(aggp, x, W1l.T, W1r.T, b1l.reshape(1, -1))
    pq = _tc_bn_proj(hpre, civ, stats, gamma1.reshape(1, -1),
                     beta1.reshape(1, -1), w2t)

    zeros2 = jnp.zeros((NP, PQ_D), jnp.float32)
    agg2p = _sc_aggregate_small(pq, src_p, dst_p, zeros2, PQ_D)

    return _tc_final(agg2p, pq, b2l.reshape(1, -1))


# speedup vs baseline: 2.0124x; 1.1212x over previous
import functools

import jax
import jax.numpy as jnp
from jax import lax
from jax.experimental import pallas as pl
from jax.experimental.pallas import tpu as pltpu
from jax.experimental.pallas import tpu_sc as plsc

N = 10000
E = 320000
IN_DIM = 128
HID_DIM = 256
OUT_DIM = 2

XA_D = 144
PQ_D = 16
NC = 2
NS = 16
EPT = E // (NC * NS)
CH = 128
NCH = 80
EPT_P = NCH * CH
NP = N + 8
NB2 = 8
NR2 = NCH // NB2
CH1 = 40
NCHUNK1 = EPT // CH1
NSEG1 = 5
NCH_SEG1 = NCHUNK1 // NSEG1
NB1 = 5
NR1 = NCH_SEG1 // NB1


def _sc_aggregate(table, src, dst, zeros, d):
    mesh = plsc.VectorSubcoreMesh(core_axis_name="core", subcore_axis_name="subcore")

    @functools.partial(
        pl.kernel,
        out_type=jax.ShapeDtypeStruct((NC, N, d), jnp.float32),
        mesh=mesh,
        compiler_params=pltpu.CompilerParams(use_tc_tiling_on_sc=False),
        scratch_types=(
            [pltpu.VMEM((NCH_SEG1, CH1), jnp.int32)] * 2
            + [pltpu.VMEM((CH1, d), jnp.float32)] * NB1
            + [pltpu.SemaphoreType.DMA] * (2 * NB1)
            + [pltpu.VMEM_SHARED((N, d), jnp.float32)]
        ),
    )
    def agg_kernel(table_hbm, src_hbm, dst_hbm, zeros_hbm, out_hbm, *scr):
        srcs, dsts = scr[0], scr[1]
        rows = scr[2:2 + NB1]
        gsem = scr[2 + NB1:2 + 2 * NB1]
        ssem = scr[2 + 2 * NB1:2 + 3 * NB1]
        acc = scr[2 + 3 * NB1]
        c = lax.axis_index("core")
        s = lax.axis_index("subcore")

        @pl.when(s == 0)
        def _():
            pltpu.sync_copy(zeros_hbm, acc)

        plsc.subcore_barrier()

        @pl.loop(0, NSEG1)
        def _(g):
            pltpu.sync_copy(src_hbm.at[c, s, g], srcs)
            pltpu.sync_copy(dst_hbm.at[c, s, g], dsts)

            for j in range(NB1):
                pltpu.async_copy(table_hbm.at[srcs.at[j]], rows[j], gsem[j])

            @pl.loop(0, NR1)
            def _(r):
                base = r * NB1
                for j in range(NB1):
                    pltpu.make_async_copy(table_hbm.at[srcs.at[base + j]],
                                          rows[j], gsem[j]).wait()
                    pltpu.async_copy(rows[j], acc.at[dsts.at[base + j]],
                                     ssem[j], add=True)

                @pl.when(r < NR1 - 1)
                def _():
                    for j in range(NB1):
                        pltpu.make_async_copy(rows[j],
                                              acc.at[dsts.at[base + j]],
                                              ssem[j]).wait()
                        pltpu.async_copy(table_hbm.at[srcs.at[base + NB1 + j]],
                                         rows[j], gsem[j])

            for j in range(NB1):
                pltpu.make_async_copy(rows[j],
                                      acc.at[dsts.at[NCH_SEG1 - NB1 + j]],
                                      ssem[j]).wait()

        plsc.subcore_barrier()

        rpt = 624
        r0 = s * rpt
        pltpu.sync_copy(acc.at[pl.ds(r0, rpt)], out_hbm.at[c, pl.ds(r0, rpt)])

        @pl.when(s == 0)
        def _():
            tail = NS * rpt
            pltpu.sync_copy(acc.at[pl.ds(tail, N - tail)],
                            out_hbm.at[c, pl.ds(tail, N - tail)])

    return agg_kernel(table, src, dst, zeros)


def _sc_aggregate_small(table, src, dst, zeros, d):
    mesh = plsc.VectorSubcoreMesh(core_axis_name="core", subcore_axis_name="subcore")

    @functools.partial(
        pl.kernel,
        out_type=jax.ShapeDtypeStruct((NC, NP, d), jnp.float32),
        mesh=mesh,
        compiler_params=pltpu.CompilerParams(use_tc_tiling_on_sc=False),
        scratch_types=(
            [pltpu.VMEM((NCH, CH), jnp.int32)] * 2
            + [pltpu.VMEM((CH, d), jnp.float32)] * NB2
            + [pltpu.SemaphoreType.DMA] * (2 * NB2)
            + [pltpu.VMEM_SHARED((NP, d), jnp.float32)]
        ),
    )
    def agg_kernel(table_hbm, src_hbm, dst_hbm, zeros_hbm, out_hbm, *scr):
        srcs, dsts = scr[0], scr[1]
        rows = scr[2:2 + NB2]
        gsem = scr[2 + NB2:2 + 2 * NB2]
        ssem = scr[2 + 2 * NB2:2 + 3 * NB2]
        acc = scr[2 + 3 * NB2]
        c = lax.axis_index("core")
        s = lax.axis_index("subcore")

        @pl.when(s == 0)
        def _():
            pltpu.sync_copy(zeros_hbm, acc)

        pltpu.sync_copy(src_hbm.at[c, s], srcs)
        pltpu.sync_copy(dst_hbm.at[c, s], dsts)

        plsc.subcore_barrier()

        for j in range(NB2):
            pltpu.async_copy(table_hbm.at[srcs.at[j]], rows[j], gsem[j])

        @pl.loop(0, NR2)
        def _(r):
            base = r * NB2
            for j in range(NB2):
                pltpu.make_async_copy(table_hbm.at[srcs.at[base + j]],
                                      rows[j], gsem[j]).wait()
                pltpu.async_copy(rows[j], acc.at[dsts.at[base + j]], ssem[j],
                                 add=True)

            @pl.when(r < NR2 - 1)
            def _():
                for j in range(NB2):
                    pltpu.make_async_copy(rows[j], acc.at[dsts.at[base + j]],
                                          ssem[j]).wait()
                    pltpu.async_copy(table_hbm.at[srcs.at[base + NB2 + j]],
                                     rows[j], gsem[j])

        for j in range(NB2):
            pltpu.make_async_copy(rows[j], acc.at[dsts.at[NCH - NB2 + j]],
                                  ssem[j]).wait()

        plsc.subcore_barrier()

        rpt = 624
        r0 = s * rpt
        pltpu.sync_copy(acc.at[pl.ds(r0, rpt)], out_hbm.at[c, pl.ds(r0, rpt)])

        @pl.when(s == 0)
        def _():
            tail = NS * rpt
            pltpu.sync_copy(acc.at[pl.ds(tail, NP - tail)],
                            out_hbm.at[c, pl.ds(tail, NP - tail)])

    return agg_kernel(table, src, dst, zeros)


BR = 2000
NBR = N // BR


def _dot(a, b):
    return lax.dot_general(a, b, (((1,), (0,)), ((), ())),
                           preferred_element_type=jnp.float32,
                           precision=lax.Precision.DEFAULT)


def _tc_sage1(aggp, x, w1lt, w1rt, b1l2):
    def body(ap_ref, x_ref, w1l_ref, w1r_ref, b1l_ref,
             hpre_ref, civ_ref, stats_ref):
        i = pl.program_id(0)
        agg = ap_ref[0] + ap_ref[1]
        civ = 1.0 / jnp.maximum(agg[:, IN_DIM:IN_DIM + 1], 1.0)
        mean = agg[:, :IN_DIM] * civ
        h = _dot(mean, w1l_ref[...]) + _dot(x_ref[...], w1r_ref[...]) + b1l_ref[...]
        hpre_ref[...] = h
        civ_ref[...] = civ

        @pl.when(i == 0)
        def _():
            stats_ref[...] = jnp.zeros_like(stats_ref)

        stats_ref[0:1, :] += jnp.sum(h, axis=0, keepdims=True)
        stats_ref[1:2, :] += jnp.sum(h * h, axis=0, keepdims=True)

    return pl.pallas_call(
        body,
        grid=(NBR,),
        in_specs=[
            pl.BlockSpec((NC, BR, XA_D), lambda i: (0, i, 0)),
            pl.BlockSpec((BR, IN_DIM), lambda i: (i, 0)),
            pl.BlockSpec((IN_DIM, HID_DIM), lambda i: (0, 0)),
            pl.BlockSpec((IN_DIM, HID_DIM), lambda i: (0, 0)),
            pl.BlockSpec((1, HID_DIM), lambda i: (0, 0)),
        ],
        out_specs=[
            pl.BlockSpec((BR, HID_DIM), lambda i: (i, 0)),
            pl.BlockSpec((BR, 1), lambda i: (i, 0)),
            pl.BlockSpec((2, HID_DIM), lambda i: (0, 0)),
        ],
        out_shape=[
            jax.ShapeDtypeStruct((N, HID_DIM), jnp.float32),
            jax.ShapeDtypeStruct((N, 1), jnp.float32),
            jax.ShapeDtypeStruct((2, HID_DIM), jnp.float32),
        ],
    )(aggp, x, w1lt, w1rt, b1l2)


def _tc_bn_proj(hpre, civ, stats, gamma2, beta2, w2t):
    def body(h_ref, civ_ref, stats_ref, g_ref, bta_ref, w2t_ref, pq_ref):
        mu = stats_ref[0:1, :] * (1.0 / N)
        var = stats_ref[1:2, :] * (1.0 / N) - mu * mu
        h = (h_ref[...] - mu) * lax.rsqrt(var + 1e-5) * g_ref[...] + bta_ref[...]
        h = jnp.maximum(h, 0.0)
        pq = _dot(h, w2t_ref[...])
        pq_ref[...] = jnp.concatenate(
            [pq[:, :2 * OUT_DIM], civ_ref[...], pq[:, 2 * OUT_DIM + 1:]], axis=1)

    return pl.pallas_call(
        body,
        grid=(NBR,),
        in_specs=[
            pl.BlockSpec((BR, HID_DIM), lambda i: (i, 0)),
            pl.BlockSpec((BR, 1), lambda i: (i, 0)),
            pl.BlockSpec((2, HID_DIM), lambda i: (0, 0)),
            pl.BlockSpec((1, HID_DIM), lambda i: (0, 0)),
            pl.BlockSpec((1, HID_DIM), lambda i: (0, 0)),
            pl.BlockSpec((HID_DIM, PQ_D), lambda i: (0, 0)),
        ],
        out_specs=pl.BlockSpec((BR, PQ_D), lambda i: (i, 0)),
        out_shape=jax.ShapeDtypeStruct((N, PQ_D), jnp.float32),
    )(hpre, civ, stats, gamma2, beta2, w2t)


def _tc_final(agg2p, pq, b2l2):
    def body(a_ref, pq_ref, b_ref, out_ref):
        a = a_ref[0] + a_ref[1]
        meanp = a[:, :OUT_DIM] * pq_ref[:, 2 * OUT_DIM:2 * OUT_DIM + 1]
        out_ref[...] = meanp + pq_ref[:, OUT_DIM:2 * OUT_DIM] + b_ref[...]

    return pl.pallas_call(
        body,
        grid=(NBR,),
        in_specs=[
            pl.BlockSpec((NC, BR, PQ_D), lambda i: (0, i, 0)),
            pl.BlockSpec((BR, PQ_D), lambda i: (i, 0)),
            pl.BlockSpec((1, OUT_DIM), lambda i: (0, 0)),
        ],
        out_specs=pl.BlockSpec((BR, OUT_DIM), lambda i: (i, 0)),
        out_shape=jax.ShapeDtypeStruct((N, OUT_DIM), jnp.float32),
    )(agg2p, pq, b2l2)


def kernel(x, edge_index, W1l, b1l, W1r, gamma1, beta1, W2l, b2l, W2r):
    pad_s = jnp.zeros((NC * NS, EPT_P - EPT), jnp.int32)
    pad_d = jnp.full((NC * NS, EPT_P - EPT), N, jnp.int32)
    src_p = jnp.concatenate([edge_index[0].reshape(NC * NS, EPT), pad_s],
                            axis=1).reshape(NC, NS, NCH, CH)
    dst_p = jnp.concatenate([edge_index[1].reshape(NC * NS, EPT), pad_d],
                            axis=1).reshape(NC, NS, NCH, CH)

    src_5 = edge_index[0].reshape(NC, NS, NSEG1, NCH_SEG1, CH1)
    dst_5 = edge_index[1].reshape(NC, NS, NSEG1, NCH_SEG1, CH1)
    xa = jnp.concatenate(
        [x, jnp.ones((N, 1), jnp.float32),
         jnp.zeros((N, XA_D - IN_DIM - 1), jnp.float32)], axis=1)
    zeros1 = jnp.zeros((N, XA_D), jnp.float32)
    aggp = _sc_aggregate(xa, src_5, dst_5, zeros1, XA_D)

    w2t = jnp.concatenate(
        [W2l.T, W2r.T, jnp.zeros((HID_DIM, PQ_D - 2 * OUT_DIM), jnp.float32)],
        axis=1)
    hpre, civ, stats = _tc_sage1(aggp, x, W1l.T, W1r.T, b1l.reshape(1, -1))
    pq = _tc_bn_proj(hpre, civ, stats, gamma1.reshape(1, -1),
                     beta1.reshape(1, -1), w2t)

    zeros2 = jnp.zeros((NP, PQ_D), jnp.float32)
    agg2p = _sc_aggregate_small(pq, src_p, dst_p, zeros2, PQ_D)

    return _tc_final(agg2p, pq, b2l.reshape(1, -1))


# speedup vs baseline: 2.2317x; 1.1090x over previous
import functools

import jax
import jax.numpy as jnp
from jax import lax
from jax.experimental import pallas as pl
from jax.experimental.pallas import tpu as pltpu
from jax.experimental.pallas import tpu_sc as plsc

N = 10000
E = 320000
IN_DIM = 128
HID_DIM = 256
OUT_DIM = 2

PQ_D = 16
NC = 2
NS = 16
EPT = E // (NC * NS)
CH = 128
NCH = 80
EPT_P = NCH * CH
NP = N + 8
NB2 = 8
NR2 = NCH // NB2
CH1 = 40
NCHUNK1 = EPT // CH1
NSEG1 = 5
NCH_SEG1 = NCHUNK1 // NSEG1
NB1 = 5
NR1 = NCH_SEG1 // NB1


CNT_D = 16


def _sc_aggregate(table, src, dst, zeros, zeros_c, ones_c, d):
    mesh = plsc.VectorSubcoreMesh(core_axis_name="core", subcore_axis_name="subcore")

    @functools.partial(
        pl.kernel,
        out_type=[jax.ShapeDtypeStruct((NC, N, d), jnp.float32),
                  jax.ShapeDtypeStruct((NC, N, CNT_D), jnp.float32)],
        mesh=mesh,
        compiler_params=pltpu.CompilerParams(use_tc_tiling_on_sc=False),
        scratch_types=(
            [pltpu.VMEM((NCH_SEG1, CH1), jnp.int32)] * 2
            + [pltpu.VMEM((CH1, d), jnp.float32)] * NB1
            + [pltpu.VMEM((CH1, CNT_D), jnp.float32)]
            + [pltpu.SemaphoreType.DMA] * (2 * NB1 + 1)
            + [pltpu.VMEM_SHARED((N, d), jnp.float32),
               pltpu.VMEM_SHARED((N, CNT_D), jnp.float32)]
        ),
    )
    def agg_kernel(table_hbm, src_hbm, dst_hbm, zeros_hbm, zc_hbm, ones_hbm,
                   out_hbm, cnt_hbm, *scr):
        srcs, dsts = scr[0], scr[1]
        rows = scr[2:2 + NB1]
        ones_v = scr[2 + NB1]
        gsem = scr[3 + NB1:3 + 2 * NB1]
        ssem = scr[3 + 2 * NB1:3 + 3 * NB1]
        csem = scr[3 + 3 * NB1]
        acc = scr[4 + 3 * NB1]
        acc_c = scr[5 + 3 * NB1]
        c = lax.axis_index("core")
        s = lax.axis_index("subcore")

        @pl.when(s == 0)
        def _():
            pltpu.sync_copy(zeros_hbm, acc)
            pltpu.sync_copy(zc_hbm, acc_c)

        pltpu.sync_copy(ones_hbm, ones_v)

        plsc.subcore_barrier()

        @pl.loop(0, NSEG1)
        def _(g):
            pltpu.sync_copy(src_hbm.at[c, s, g], srcs)
            pltpu.sync_copy(dst_hbm.at[c, s, g], dsts)

            for j in range(NB1):
                pltpu.async_copy(table_hbm.at[srcs.at[j]], rows[j], gsem[j])

            @pl.loop(0, NR1)
            def _(r):
                base = r * NB1
                for j in range(NB1):
                    pltpu.make_async_copy(table_hbm.at[srcs.at[base + j]],
                                          rows[j], gsem[j]).wait()
                    pltpu.async_copy(rows[j], acc.at[dsts.at[base + j]],
                                     ssem[j], add=True)
                    pltpu.async_copy(ones_v, acc_c.at[dsts.at[base + j]],
                                     csem, add=True)

                @pl.when(r < NR1 - 1)
                def _():
                    for j in range(NB1):
                        pltpu.make_async_copy(rows[j],
                                              acc.at[dsts.at[base + j]],
                                              ssem[j]).wait()
                        pltpu.async_copy(table_hbm.at[srcs.at[base + NB1 + j]],
                                         rows[j], gsem[j])

            for j in range(NB1):
                pltpu.make_async_copy(rows[j],
                                      acc.at[dsts.at[NCH_SEG1 - NB1 + j]],
                                      ssem[j]).wait()

            @pl.loop(0, NCH_SEG1)
            def _(i):
                pltpu.make_async_copy(ones_v, acc_c.at[dsts.at[i]], csem).wait()

        plsc.subcore_barrier()

        rpt = 624
        r0 = s * rpt
        pltpu.sync_copy(acc.at[pl.ds(r0, rpt)], out_hbm.at[c, pl.ds(r0, rpt)])
        pltpu.sync_copy(acc_c.at[pl.ds(r0, rpt)], cnt_hbm.at[c, pl.ds(r0, rpt)])

        @pl.when(s == 0)
        def _():
            tail = NS * rpt
            pltpu.sync_copy(acc.at[pl.ds(tail, N - tail)],
                            out_hbm.at[c, pl.ds(tail, N - tail)])
            pltpu.sync_copy(acc_c.at[pl.ds(tail, N - tail)],
                            cnt_hbm.at[c, pl.ds(tail, N - tail)])

    return agg_kernel(table, src, dst, zeros, zeros_c, ones_c)


def _sc_aggregate_small(table, src, dst, zeros, d):
    mesh = plsc.VectorSubcoreMesh(core_axis_name="core", subcore_axis_name="subcore")

    @functools.partial(
        pl.kernel,
        out_type=jax.ShapeDtypeStruct((NC, NP, d), jnp.float32),
        mesh=mesh,
        compiler_params=pltpu.CompilerParams(use_tc_tiling_on_sc=False),
        scratch_types=(
            [pltpu.VMEM((NCH, CH), jnp.int32)] * 2
            + [pltpu.VMEM((CH, d), jnp.float32)] * NB2
            + [pltpu.SemaphoreType.DMA] * (2 * NB2)
            + [pltpu.VMEM_SHARED((NP, d), jnp.float32)]
        ),
    )
    def agg_kernel(table_hbm, src_hbm, dst_hbm, zeros_hbm, out_hbm, *scr):
        srcs, dsts = scr[0], scr[1]
        rows = scr[2:2 + NB2]
        gsem = scr[2 + NB2:2 + 2 * NB2]
        ssem = scr[2 + 2 * NB2:2 + 3 * NB2]
        acc = scr[2 + 3 * NB2]
        c = lax.axis_index("core")
        s = lax.axis_index("subcore")

        @pl.when(s == 0)
        def _():
            pltpu.sync_copy(zeros_hbm, acc)

        pltpu.sync_copy(src_hbm.at[c, s], srcs)
        pltpu.sync_copy(dst_hbm.at[c, s], dsts)

        plsc.subcore_barrier()

        for j in range(NB2):
            pltpu.async_copy(table_hbm.at[srcs.at[j]], rows[j], gsem[j])

        @pl.loop(0, NR2)
        def _(r):
            base = r * NB2
            for j in range(NB2):
                pltpu.make_async_copy(table_hbm.at[srcs.at[base + j]],
                                      rows[j], gsem[j]).wait()
                pltpu.async_copy(rows[j], acc.at[dsts.at[base + j]], ssem[j],
                                 add=True)

            @pl.when(r < NR2 - 1)
            def _():
                for j in range(NB2):
                    pltpu.make_async_copy(rows[j], acc.at[dsts.at[base + j]],
                                          ssem[j]).wait()
                    pltpu.async_copy(table_hbm.at[srcs.at[base + NB2 + j]],
                                     rows[j], gsem[j])

        for j in range(NB2):
            pltpu.make_async_copy(rows[j], acc.at[dsts.at[NCH - NB2 + j]],
                                  ssem[j]).wait()

        plsc.subcore_barrier()

        rpt = 624
        r0 = s * rpt
        pltpu.sync_copy(acc.at[pl.ds(r0, rpt)], out_hbm.at[c, pl.ds(r0, rpt)])

        @pl.when(s == 0)
        def _():
            tail = NS * rpt
            pltpu.sync_copy(acc.at[pl.ds(tail, NP - tail)],
                            out_hbm.at[c, pl.ds(tail, NP - tail)])

    return agg_kernel(table, src, dst, zeros)


BR = 2000
NBR = N // BR


def _dot(a, b):
    return lax.dot_general(a, b, (((1,), (0,)), ((), ())),
                           preferred_element_type=jnp.float32,
                           precision=lax.Precision.DEFAULT)


def _tc_sage1(aggf, aggc, x, w1lt, w1rt, b1l2):
    def body(af_ref, ac_ref, x_ref, w1l_ref, w1r_ref, b1l_ref,
             hpre_ref, civ_ref, stats_ref):
        i = pl.program_id(0)
        agg = af_ref[0] + af_ref[1]
        cnt = ac_ref[0, :, 0:1] + ac_ref[1, :, 0:1]
        civ = 1.0 / jnp.maximum(cnt, 1.0)
        mean = agg * civ
        h = _dot(mean, w1l_ref[...]) + _dot(x_ref[...], w1r_ref[...]) + b1l_ref[...]
        hpre_ref[...] = h
        civ_ref[...] = civ

        @pl.when(i == 0)
        def _():
            stats_ref[...] = jnp.zeros_like(stats_ref)

        stats_ref[0:1, :] += jnp.sum(h, axis=0, keepdims=True)
        stats_ref[1:2, :] += jnp.sum(h * h, axis=0, keepdims=True)

    return pl.pallas_call(
        body,
        grid=(NBR,),
        in_specs=[
            pl.BlockSpec((NC, BR, IN_DIM), lambda i: (0, i, 0)),
            pl.BlockSpec((NC, BR, CNT_D), lambda i: (0, i, 0)),
            pl.BlockSpec((BR, IN_DIM), lambda i: (i, 0)),
            pl.BlockSpec((IN_DIM, HID_DIM), lambda i: (0, 0)),
            pl.BlockSpec((IN_DIM, HID_DIM), lambda i: (0, 0)),
            pl.BlockSpec((1, HID_DIM), lambda i: (0, 0)),
        ],
        out_specs=[
            pl.BlockSpec((BR, HID_DIM), lambda i: (i, 0)),
            pl.BlockSpec((BR, 1), lambda i: (i, 0)),
            pl.BlockSpec((2, HID_DIM), lambda i: (0, 0)),
        ],
        out_shape=[
            jax.ShapeDtypeStruct((N, HID_DIM), jnp.float32),
            jax.ShapeDtypeStruct((N, 1), jnp.float32),
            jax.ShapeDtypeStruct((2, HID_DIM), jnp.float32),
        ],
    )(aggf, aggc, x, w1lt, w1rt, b1l2)


def _tc_bn_proj(hpre, civ, stats, gamma2, beta2, w2t):
    def body(h_ref, civ_ref, stats_ref, g_ref, bta_ref, w2t_ref, pq_ref):
        mu = stats_ref[0:1, :] * (1.0 / N)
        var = stats_ref[1:2, :] * (1.0 / N) - mu * mu
        h = (h_ref[...] - mu) * lax.rsqrt(var + 1e-5) * g_ref[...] + bta_ref[...]
        h = jnp.maximum(h, 0.0)
        pq = _dot(h, w2t_ref[...])
        pq_ref[...] = jnp.concatenate(
            [pq[:, :2 * OUT_DIM], civ_ref[...], pq[:, 2 * OUT_DIM + 1:]], axis=1)

    return pl.pallas_call(
        body,
        grid=(NBR,),
        in_specs=[
            pl.BlockSpec((BR, HID_DIM), lambda i: (i, 0)),
            pl.BlockSpec((BR, 1), lambda i: (i, 0)),
            pl.BlockSpec((2, HID_DIM), lambda i: (0, 0)),
            pl.BlockSpec((1, HID_DIM), lambda i: (0, 0)),
            pl.BlockSpec((1, HID_DIM), lambda i: (0, 0)),
            pl.BlockSpec((HID_DIM, PQ_D), lambda i: (0, 0)),
        ],
        out_specs=pl.BlockSpec((BR, PQ_D), lambda i: (i, 0)),
        out_shape=jax.ShapeDtypeStruct((N, PQ_D), jnp.float32),
    )(hpre, civ, stats, gamma2, beta2, w2t)


def _tc_final(agg2p, pq, b2l2):
    def body(a_ref, pq_ref, b_ref, out_ref):
        a = a_ref[0] + a_ref[1]
        meanp = a[:, :OUT_DIM] * pq_ref[:, 2 * OUT_DIM:2 * OUT_DIM + 1]
        out_ref[...] = meanp + pq_ref[:, OUT_DIM:2 * OUT_DIM] + b_ref[...]

    return pl.pallas_call(
        body,
        grid=(NBR,),
        in_specs=[
            pl.BlockSpec((NC, BR, PQ_D), lambda i: (0, i, 0)),
            pl.BlockSpec((BR, PQ_D), lambda i: (i, 0)),
            pl.BlockSpec((1, OUT_DIM), lambda i: (0, 0)),
        ],
        out_specs=pl.BlockSpec((BR, OUT_DIM), lambda i: (i, 0)),
        out_shape=jax.ShapeDtypeStruct((N, OUT_DIM), jnp.float32),
    )(agg2p, pq, b2l2)


def kernel(x, edge_index, W1l, b1l, W1r, gamma1, beta1, W2l, b2l, W2r):
    pad_s = jnp.zeros((NC * NS, EPT_P - EPT), jnp.int32)
    pad_d = jnp.full((NC * NS, EPT_P - EPT), N, jnp.int32)
    src_p = jnp.concatenate([edge_index[0].reshape(NC * NS, EPT), pad_s],
                            axis=1).reshape(NC, NS, NCH, CH)
    dst_p = jnp.concatenate([edge_index[1].reshape(NC * NS, EPT), pad_d],
                            axis=1).reshape(NC, NS, NCH, CH)

    src_5 = edge_index[0].reshape(NC, NS, NSEG1, NCH_SEG1, CH1)
    dst_5 = edge_index[1].reshape(NC, NS, NSEG1, NCH_SEG1, CH1)
    zeros1 = jnp.zeros((N, IN_DIM), jnp.float32)
    zeros_c = jnp.zeros((N, CNT_D), jnp.float32)
    ones_c = jnp.ones((CH1, CNT_D), jnp.float32)
    aggf, aggc = _sc_aggregate(x, src_5, dst_5, zeros1, zeros_c, ones_c, IN_DIM)

    w2t = jnp.concatenate(
        [W2l.T, W2r.T, jnp.zeros((HID_DIM, PQ_D - 2 * OUT_DIM), jnp.float32)],
        axis=1)
    hpre, civ, stats = _tc_sage1(aggf, aggc, x, W1l.T, W1r.T, b1l.reshape(1, -1))
    pq = _tc_bn_proj(hpre, civ, stats, gamma1.reshape(1, -1),
                     beta1.reshape(1, -1), w2t)

    zeros2 = jnp.zeros((NP, PQ_D), jnp.float32)
    agg2p = _sc_aggregate_small(pq, src_p, dst_p, zeros2, PQ_D)

    return _tc_final(agg2p, pq, b2l.reshape(1, -1))


# speedup vs baseline: 2.3130x; 1.0364x over previous
import functools

import jax
import jax.numpy as jnp
from jax import lax
from jax.experimental import pallas as pl
from jax.experimental.pallas import tpu as pltpu
from jax.experimental.pallas import tpu_sc as plsc

N = 10000
E = 320000
IN_DIM = 128
HID_DIM = 256
OUT_DIM = 2

PQ_D = 16
NC = 2
NS = 16
EPT = E // (NC * NS)
CH = 128
NCH = 80
EPT_P = NCH * CH
NP = N + 8
NB2 = 8
NR2 = NCH // NB2
CH1 = 40
NCHUNK1 = EPT // CH1
NSEG1 = 5
NCH_SEG1 = NCHUNK1 // NSEG1
NB1 = 5
NR1 = NCH_SEG1 // NB1


CNT_D = 16


def _sc_aggregate(table, edges, zeros, zeros_c, ones_c, d):
    mesh = plsc.VectorSubcoreMesh(core_axis_name="core", subcore_axis_name="subcore")

    @functools.partial(
        pl.kernel,
        out_type=[jax.ShapeDtypeStruct((NC, N, d), jnp.float32),
                  jax.ShapeDtypeStruct((NC, N, CNT_D), jnp.float32)],
        mesh=mesh,
        compiler_params=pltpu.CompilerParams(use_tc_tiling_on_sc=False),
        scratch_types=(
            [pltpu.VMEM((NCH_SEG1, CH1), jnp.int32)] * 2
            + [pltpu.VMEM((CH1, d), jnp.float32)] * NB1
            + [pltpu.VMEM((CH1, CNT_D), jnp.float32)]
            + [pltpu.SemaphoreType.DMA] * (2 * NB1 + 1)
            + [pltpu.VMEM_SHARED((N, d), jnp.float32),
               pltpu.VMEM_SHARED((N, CNT_D), jnp.float32)]
        ),
    )
    def agg_kernel(table_hbm, edge_hbm, zeros_hbm, zc_hbm, ones_hbm,
                   out_hbm, cnt_hbm, *scr):
        srcs, dsts = scr[0], scr[1]
        rows = scr[2:2 + NB1]
        ones_v = scr[2 + NB1]
        gsem = scr[3 + NB1:3 + 2 * NB1]
        ssem = scr[3 + 2 * NB1:3 + 3 * NB1]
        csem = scr[3 + 3 * NB1]
        acc = scr[4 + 3 * NB1]
        acc_c = scr[5 + 3 * NB1]
        c = lax.axis_index("core")
        s = lax.axis_index("subcore")

        @pl.when(s == 0)
        def _():
            pltpu.sync_copy(zeros_hbm, acc)
            pltpu.sync_copy(zc_hbm, acc_c)

        pltpu.sync_copy(ones_hbm, ones_v)

        plsc.subcore_barrier()

        @pl.loop(0, NSEG1)
        def _(g):
            pltpu.sync_copy(edge_hbm.at[0, c, s, g], srcs)
            pltpu.sync_copy(edge_hbm.at[1, c, s, g], dsts)

            for j in range(NB1):
                pltpu.async_copy(table_hbm.at[srcs.at[j]], rows[j], gsem[j])

            @pl.loop(0, NR1)
            def _(r):
                base = r * NB1
                for j in range(NB1):
                    pltpu.make_async_copy(table_hbm.at[srcs.at[base + j]],
                                          rows[j], gsem[j]).wait()
                    pltpu.async_copy(rows[j], acc.at[dsts.at[base + j]],
                                     ssem[j], add=True)
                    pltpu.async_copy(ones_v, acc_c.at[dsts.at[base + j]],
                                     csem, add=True)

                @pl.when(r < NR1 - 1)
                def _():
                    for j in range(NB1):
                        pltpu.make_async_copy(rows[j],
                                              acc.at[dsts.at[base + j]],
                                              ssem[j]).wait()
                        pltpu.async_copy(table_hbm.at[srcs.at[base + NB1 + j]],
                                         rows[j], gsem[j])

            for j in range(NB1):
                pltpu.make_async_copy(rows[j],
                                      acc.at[dsts.at[NCH_SEG1 - NB1 + j]],
                                      ssem[j]).wait()

            @pl.loop(0, NCH_SEG1)
            def _(i):
                pltpu.make_async_copy(ones_v, acc_c.at[dsts.at[i]], csem).wait()

        plsc.subcore_barrier()

        rpt = 624
        r0 = s * rpt
        pltpu.sync_copy(acc.at[pl.ds(r0, rpt)], out_hbm.at[c, pl.ds(r0, rpt)])
        pltpu.sync_copy(acc_c.at[pl.ds(r0, rpt)], cnt_hbm.at[c, pl.ds(r0, rpt)])

        @pl.when(s == 0)
        def _():
            tail = NS * rpt
            pltpu.sync_copy(acc.at[pl.ds(tail, N - tail)],
                            out_hbm.at[c, pl.ds(tail, N - tail)])
            pltpu.sync_copy(acc_c.at[pl.ds(tail, N - tail)],
                            cnt_hbm.at[c, pl.ds(tail, N - tail)])

    return agg_kernel(table, edges, zeros, zeros_c, ones_c)


def _sc_aggregate_small(table, src, dst, zeros, d):
    mesh = plsc.VectorSubcoreMesh(core_axis_name="core", subcore_axis_name="subcore")

    @functools.partial(
        pl.kernel,
        out_type=jax.ShapeDtypeStruct((NC, NP, d), jnp.float32),
        mesh=mesh,
        compiler_params=pltpu.CompilerParams(use_tc_tiling_on_sc=False),
        scratch_types=(
            [pltpu.VMEM((NCH, CH), jnp.int32)] * 2
            + [pltpu.VMEM((CH, d), jnp.float32)] * NB2
            + [pltpu.SemaphoreType.DMA] * (2 * NB2)
            + [pltpu.VMEM_SHARED((NP, d), jnp.float32)]
        ),
    )
    def agg_kernel(table_hbm, src_hbm, dst_hbm, zeros_hbm, out_hbm, *scr):
        srcs, dsts = scr[0], scr[1]
        rows = scr[2:2 + NB2]
        gsem = scr[2 + NB2:2 + 2 * NB2]
        ssem = scr[2 + 2 * NB2:2 + 3 * NB2]
        acc = scr[2 + 3 * NB2]
        c = lax.axis_index("core")
        s = lax.axis_index("subcore")

        @pl.when(s == 0)
        def _():
            pltpu.sync_copy(zeros_hbm, acc)

        pltpu.sync_copy(src_hbm.at[c, s], srcs)
        pltpu.sync_copy(dst_hbm.at[c, s], dsts)

        plsc.subcore_barrier()

        for j in range(NB2):
            pltpu.async_copy(table_hbm.at[srcs.at[j]], rows[j], gsem[j])

        @pl.loop(0, NR2)
        def _(r):
            base = r * NB2
            for j in range(NB2):
                pltpu.make_async_copy(table_hbm.at[srcs.at[base + j]],
                                      rows[j], gsem[j]).wait()
                pltpu.async_copy(rows[j], acc.at[dsts.at[base + j]], ssem[j],
                                 add=True)

            @pl.when(r < NR2 - 1)
            def _():
                for j in range(NB2):
                    pltpu.make_async_copy(rows[j], acc.at[dsts.at[base + j]],
                                          ssem[j]).wait()
                    pltpu.async_copy(table_hbm.at[srcs.at[base + NB2 + j]],
                                     rows[j], gsem[j])

        for j in range(NB2):
            pltpu.make_async_copy(rows[j], acc.at[dsts.at[NCH - NB2 + j]],
                                  ssem[j]).wait()

        plsc.subcore_barrier()

        rpt = 624
        r0 = s * rpt
        pltpu.sync_copy(acc.at[pl.ds(r0, rpt)], out_hbm.at[c, pl.ds(r0, rpt)])

        @pl.when(s == 0)
        def _():
            tail = NS * rpt
            pltpu.sync_copy(acc.at[pl.ds(tail, NP - tail)],
                            out_hbm.at[c, pl.ds(tail, NP - tail)])

    return agg_kernel(table, src, dst, zeros)


BR = 2000
NBR = N // BR


def _dot(a, b):
    return lax.dot_general(a, b, (((1,), (0,)), ((), ())),
                           preferred_element_type=jnp.float32,
                           precision=lax.Precision.DEFAULT)


def _tc_sage1(aggf, aggc, x, w1lt, w1rt, b1l2):
    def body(af_ref, ac_ref, x_ref, w1l_ref, w1r_ref, b1l_ref,
             hpre_ref, civ_ref, stats_ref):
        i = pl.program_id(0)
        agg = af_ref[0] + af_ref[1]
        cnt = ac_ref[0, :, 0:1] + ac_ref[1, :, 0:1]
        civ = 1.0 / jnp.maximum(cnt, 1.0)
        mean = agg * civ
        h = _dot(mean, w1l_ref[...]) + _dot(x_ref[...], w1r_ref[...]) + b1l_ref[...]
        hpre_ref[...] = h
        civ_ref[...] = civ

        @pl.when(i == 0)
        def _():
            stats_ref[...] = jnp.zeros_like(stats_ref)

        stats_ref[0:1, :] += jnp.sum(h, axis=0, keepdims=True)
        stats_ref[1:2, :] += jnp.sum(h * h, axis=0, keepdims=True)

    return pl.pallas_call(
        body,
        grid=(NBR,),
        in_specs=[
            pl.BlockSpec((NC, BR, IN_DIM), lambda i: (0, i, 0)),
            pl.BlockSpec((NC, BR, CNT_D), lambda i: (0, i, 0)),
            pl.BlockSpec((BR, IN_DIM), lambda i: (i, 0)),
            pl.BlockSpec((IN_DIM, HID_DIM), lambda i: (0, 0)),
            pl.BlockSpec((IN_DIM, HID_DIM), lambda i: (0, 0)),
            pl.BlockSpec((1, HID_DIM), lambda i: (0, 0)),
        ],
        out_specs=[
            pl.BlockSpec((BR, HID_DIM), lambda i: (i, 0)),
            pl.BlockSpec((BR, 1), lambda i: (i, 0)),
            pl.BlockSpec((2, HID_DIM), lambda i: (0, 0)),
        ],
        out_shape=[
            jax.ShapeDtypeStruct((N, HID_DIM), jnp.float32),
            jax.ShapeDtypeStruct((N, 1), jnp.float32),
            jax.ShapeDtypeStruct((2, HID_DIM), jnp.float32),
        ],
    )(aggf, aggc, x, w1lt, w1rt, b1l2)


def _tc_bn_proj(hpre, civ, stats, gamma2, beta2, w2t):
    def body(h_ref, civ_ref, stats_ref, g_ref, bta_ref, w2t_ref, pq_ref):
        mu = stats_ref[0:1, :] * (1.0 / N)
        var = stats_ref[1:2, :] * (1.0 / N) - mu * mu
        h = (h_ref[...] - mu) * lax.rsqrt(var + 1e-5) * g_ref[...] + bta_ref[...]
        h = jnp.maximum(h, 0.0)
        pq = _dot(h, w2t_ref[...])
        pq_ref[...] = jnp.concatenate(
            [pq[:, :2 * OUT_DIM], civ_ref[...], pq[:, 2 * OUT_DIM + 1:]], axis=1)

    return pl.pallas_call(
        body,
        grid=(NBR,),
        in_specs=[
            pl.BlockSpec((BR, HID_DIM), lambda i: (i, 0)),
            pl.BlockSpec((BR, 1), lambda i: (i, 0)),
            pl.BlockSpec((2, HID_DIM), lambda i: (0, 0)),
            pl.BlockSpec((1, HID_DIM), lambda i: (0, 0)),
            pl.BlockSpec((1, HID_DIM), lambda i: (0, 0)),
            pl.BlockSpec((HID_DIM, PQ_D), lambda i: (0, 0)),
        ],
        out_specs=pl.BlockSpec((BR, PQ_D), lambda i: (i, 0)),
        out_shape=jax.ShapeDtypeStruct((N, PQ_D), jnp.float32),
    )(hpre, civ, stats, gamma2, beta2, w2t)


def _tc_final(agg2p, pq, b2l2):
    def body(a_ref, pq_ref, b_ref, out_ref):
        a = a_ref[0] + a_ref[1]
        meanp = a[:, :OUT_DIM] * pq_ref[:, 2 * OUT_DIM:2 * OUT_DIM + 1]
        out_ref[...] = meanp + pq_ref[:, OUT_DIM:2 * OUT_DIM] + b_ref[...]

    return pl.pallas_call(
        body,
        grid=(NBR,),
        in_specs=[
            pl.BlockSpec((NC, BR, PQ_D), lambda i: (0, i, 0)),
            pl.BlockSpec((BR, PQ_D), lambda i: (i, 0)),
            pl.BlockSpec((1, OUT_DIM), lambda i: (0, 0)),
        ],
        out_specs=pl.BlockSpec((BR, OUT_DIM), lambda i: (i, 0)),
        out_shape=jax.ShapeDtypeStruct((N, OUT_DIM), jnp.float32),
    )(agg2p, pq, b2l2)


def kernel(x, edge_index, W1l, b1l, W1r, gamma1, beta1, W2l, b2l, W2r):
    pad_s = jnp.zeros((NC * NS, EPT_P - EPT), jnp.int32)
    pad_d = jnp.full((NC * NS, EPT_P - EPT), N, jnp.int32)
    src_p = jnp.concatenate([edge_index[0].reshape(NC * NS, EPT), pad_s],
                            axis=1).reshape(NC, NS, NCH, CH)
    dst_p = jnp.concatenate([edge_index[1].reshape(NC * NS, EPT), pad_d],
                            axis=1).reshape(NC, NS, NCH, CH)

    edges_6 = edge_index.reshape(2, NC, NS, NSEG1, NCH_SEG1, CH1)
    zeros1 = jnp.zeros((N, IN_DIM), jnp.float32)
    zeros_c = jnp.zeros((N, CNT_D), jnp.float32)
    ones_c = jnp.ones((CH1, CNT_D), jnp.float32)
    aggf, aggc = _sc_aggregate(x, edges_6, zeros1, zeros_c, ones_c, IN_DIM)

    w2t = jnp.concatenate(
        [W2l.T, W2r.T, jnp.zeros((HID_DIM, PQ_D - 2 * OUT_DIM), jnp.float32)],
        axis=1)
    hpre, civ, stats = _tc_sage1(aggf, aggc, x, W1l.T, W1r.T, b1l.reshape(1, -1))
    pq = _tc_bn_proj(hpre, civ, stats, gamma1.reshape(1, -1),
                     beta1.reshape(1, -1), w2t)

    zeros2 = jnp.zeros((NP, PQ_D), jnp.float32)
    agg2p = _sc_aggregate_small(pq, src_p, dst_p, zeros2, PQ_D)

    return _tc_final(agg2p, pq, b2l.reshape(1, -1))
